# baseline (speedup 1.0000x reference)
import jax
import jax.numpy as jnp
from jax.experimental import pallas as pl
from jax.experimental.pallas import tpu as pltpu

_BSZ = 1024
_XD = 512
_SEQ = 128
_BASE = _XD * _SEQ
_BI = 8
_GI = _BSZ // _BI
_NSPLIT = 8
_QX = _XD // _NSPLIT
_K = _BSZ // 2


def _err_kernel(*refs):
    inp_refs = refs[:_NSPLIT]
    tgt_refs = refs[_NSPLIT:2 * _NSPLIT]
    err_ref = refs[2 * _NSPLIT]
    acc = jnp.zeros((_BI, _SEQ), jnp.float32)
    for a, b in zip(inp_refs, tgt_refs):
        d = a[...] - b[...]
        acc += jnp.sum(d * d, axis=1)
    err_ref[...] = jnp.sum(acc, axis=1).reshape(1, 1, _BI) * (1.0 / _BASE)


def _loss_kernel(step_ref, err_ref, out_ref):
    errors = err_ref[0, :]
    U = jnp.mean(errors)
    var = jnp.sum((errors - U) ** 2) * (1.0 / (_BSZ - 1))
    Sigma = jnp.sqrt(var) + 1e-6
    u = 0.1 * U
    sigma = 0.1 * Sigma + 0.9
    z = jnp.abs(errors - u) * (1.0 / sigma)
    nz = -z
    e = jnp.exp(nz - jnp.max(nz))
    w1 = e * (1.0 / jnp.sum(e))
    w1 = w1 * (1.0 / jnp.mean(w1))
    col = w1.reshape(_BSZ, 1)
    row = w1.reshape(1, _BSZ)
    lt = (row < col).astype(jnp.float32)
    ji = jax.lax.broadcasted_iota(jnp.int32, (_BSZ, _BSZ), 1)
    ii = jax.lax.broadcasted_iota(jnp.int32, (_BSZ, _BSZ), 0)
    tie = ((row == col) & (ji < ii)).astype(jnp.float32)
    rank = jnp.sum(lt + tie, axis=1)
    w1m = jnp.where(rank >= _K, w1, 0.0)
    step = step_ref[0, 0]
    w = (1.0 + (step - 1.0) * w1m) * (1.0 / step)
    out_ref[...] = jnp.mean(errors * w).reshape(1, 1)


def kernel(input, target, global_step):
    step = jnp.asarray(global_step, dtype=input.dtype).reshape(1, 1)
    errors = pl.pallas_call(
        _err_kernel,
        grid=(_GI,),
        in_specs=[
            pl.BlockSpec((_BI, _QX, _SEQ), lambda i, s=s: (i, s, 0))
            for s in range(_NSPLIT)
        ] + [
            pl.BlockSpec((_BI, _QX, _SEQ), lambda i, s=s: (i, s, 0))
            for s in range(_NSPLIT)
        ],
        out_specs=pl.BlockSpec((1, 1, _BI), lambda i: (i, 0, 0)),
        out_shape=jax.ShapeDtypeStruct((_GI, 1, _BI), jnp.float32),
        compiler_params=pltpu.CompilerParams(
            dimension_semantics=("arbitrary",),
        ),
    )(*([input] * _NSPLIT + [target] * _NSPLIT)).reshape(1, _BSZ)
    out = pl.pallas_call(
        _loss_kernel,
        in_specs=[
            pl.BlockSpec((1, 1), lambda: (0, 0)),
            pl.BlockSpec((1, _BSZ), lambda: (0, 0)),
        ],
        out_specs=pl.BlockSpec((1, 1), lambda: (0, 0)),
        out_shape=jax.ShapeDtypeStruct((1, 1), jnp.float32),
    )(step, errors)
    return out[0, 0]


# speedup vs baseline: 1.1521x; 1.1521x over previous
import jax
import jax.numpy as jnp
from jax.experimental import pallas as pl
from jax.experimental.pallas import tpu as pltpu

_BSZ = 1024
_XD = 512
_SEQ = 128
_BASE = _XD * _SEQ
_BI = 16
_GI = _BSZ // _BI
_NSPLIT = 16
_QX = _XD // _NSPLIT
_K = _BSZ // 2


def _err_kernel(*refs):
    inp_refs = refs[:_NSPLIT]
    tgt_refs = refs[_NSPLIT:2 * _NSPLIT]
    err_ref = refs[2 * _NSPLIT]
    acc = jnp.zeros((_BI, _SEQ), jnp.float32)
    for a, b in zip(inp_refs, tgt_refs):
        d = a[...] - b[...]
        acc += jnp.sum(d * d, axis=1)
    err_ref[...] = jnp.sum(acc, axis=1).reshape(1, 1, _BI) * (1.0 / _BASE)


def _loss_kernel(step_ref, err_ref, out_ref):
    errors = err_ref[0, :]
    U = jnp.mean(errors)
    var = jnp.sum((errors - U) ** 2) * (1.0 / (_BSZ - 1))
    Sigma = jnp.sqrt(var) + 1e-6
    u = 0.1 * U
    sigma = 0.1 * Sigma + 0.9
    z = jnp.abs(errors - u) * (1.0 / sigma)
    nz = -z
    e = jnp.exp(nz - jnp.max(nz))
    w1 = e * (1.0 / jnp.sum(e))
    w1 = w1 * (1.0 / jnp.mean(w1))
    col = w1.reshape(_BSZ, 1)
    row = w1.reshape(1, _BSZ)
    lt = (row < col).astype(jnp.float32)
    ji = jax.lax.broadcasted_iota(jnp.int32, (_BSZ, _BSZ), 1)
    ii = jax.lax.broadcasted_iota(jnp.int32, (_BSZ, _BSZ), 0)
    tie = ((row == col) & (ji < ii)).astype(jnp.float32)
    rank = jnp.sum(lt + tie, axis=1)
    w1m = jnp.where(rank >= _K, w1, 0.0)
    step = step_ref[0, 0]
    w = (1.0 + (step - 1.0) * w1m) * (1.0 / step)
    out_ref[...] = jnp.mean(errors * w).reshape(1, 1)


def kernel(input, target, global_step):
    step = jnp.asarray(global_step, dtype=input.dtype).reshape(1, 1)
    errors = pl.pallas_call(
        _err_kernel,
        grid=(_GI,),
        in_specs=[
            pl.BlockSpec((_BI, _QX, _SEQ), lambda i, s=s: (i, s, 0))
            for s in range(_NSPLIT)
        ] + [
            pl.BlockSpec((_BI, _QX, _SEQ), lambda i, s=s: (i, s, 0))
            for s in range(_NSPLIT)
        ],
        out_specs=pl.BlockSpec((1, 1, _BI), lambda i: (i, 0, 0)),
        out_shape=jax.ShapeDtypeStruct((_GI, 1, _BI), jnp.float32),
        compiler_params=pltpu.CompilerParams(
            dimension_semantics=("arbitrary",),
        ),
    )(*([input] * _NSPLIT + [target] * _NSPLIT)).reshape(1, _BSZ)
    out = pl.pallas_call(
        _loss_kernel,
        in_specs=[
            pl.BlockSpec((1, 1), lambda: (0, 0)),
            pl.BlockSpec((1, _BSZ), lambda: (0, 0)),
        ],
        out_specs=pl.BlockSpec((1, 1), lambda: (0, 0)),
        out_shape=jax.ShapeDtypeStruct((1, 1), jnp.float32),
    )(step, errors)
    return out[0, 0]


# speedup vs baseline: 1.1536x; 1.0012x over previous
import jax
import jax.numpy as jnp
from jax.experimental import pallas as pl
from jax.experimental.pallas import tpu as pltpu

_BSZ = 1024
_XD = 512
_SEQ = 128
_BASE = _XD * _SEQ
_BI = 16
_GI = _BSZ // _BI
_NSPLIT = 4
_QX = _XD // _NSPLIT
_K = _BSZ // 2


def _err_kernel(*refs):
    inp_refs = refs[:_NSPLIT]
    tgt_refs = refs[_NSPLIT:2 * _NSPLIT]
    err_ref = refs[2 * _NSPLIT]
    acc = jnp.zeros((_BI, _SEQ), jnp.float32)
    for a, b in zip(inp_refs, tgt_refs):
        d = a[...] - b[...]
        acc += jnp.sum(d * d, axis=1)
    err_ref[...] = jnp.sum(acc, axis=1).reshape(1, 1, _BI) * (1.0 / _BASE)


def _loss_kernel(step_ref, err_ref, out_ref):
    errors = err_ref[0, :]
    U = jnp.mean(errors)
    var = jnp.sum((errors - U) ** 2) * (1.0 / (_BSZ - 1))
    Sigma = jnp.sqrt(var) + 1e-6
    u = 0.1 * U
    sigma = 0.1 * Sigma + 0.9
    z = jnp.abs(errors - u) * (1.0 / sigma)
    nz = -z
    e = jnp.exp(nz - jnp.max(nz))
    w1 = e * (1.0 / jnp.sum(e))
    w1 = w1 * (1.0 / jnp.mean(w1))
    col = w1.reshape(_BSZ, 1)
    row = w1.reshape(1, _BSZ)
    lt = (row < col).astype(jnp.float32)
    ji = jax.lax.broadcasted_iota(jnp.int32, (_BSZ, _BSZ), 1)
    ii = jax.lax.broadcasted_iota(jnp.int32, (_BSZ, _BSZ), 0)
    tie = ((row == col) & (ji < ii)).astype(jnp.float32)
    rank = jnp.sum(lt + tie, axis=1)
    w1m = jnp.where(rank >= _K, w1, 0.0)
    step = step_ref[0, 0]
    w = (1.0 + (step - 1.0) * w1m) * (1.0 / step)
    out_ref[...] = jnp.mean(errors * w).reshape(1, 1)


def kernel(input, target, global_step):
    step = jnp.asarray(global_step, dtype=input.dtype).reshape(1, 1)
    errors = pl.pallas_call(
        _err_kernel,
        grid=(_GI,),
        in_specs=[
            pl.BlockSpec((_BI, _QX, _SEQ), lambda i, s=s: (i, s, 0))
            for s in range(_NSPLIT)
        ] + [
            pl.BlockSpec((_BI, _QX, _SEQ), lambda i, s=s: (i, s, 0))
            for s in range(_NSPLIT)
        ],
        out_specs=pl.BlockSpec((1, 1, _BI), lambda i: (i, 0, 0)),
        out_shape=jax.ShapeDtypeStruct((_GI, 1, _BI), jnp.float32),
        compiler_params=pltpu.CompilerParams(
            dimension_semantics=("arbitrary",),
        ),
    )(*([input] * _NSPLIT + [target] * _NSPLIT)).reshape(1, _BSZ)
    out = pl.pallas_call(
        _loss_kernel,
        in_specs=[
            pl.BlockSpec((1, 1), lambda: (0, 0)),
            pl.BlockSpec((1, _BSZ), lambda: (0, 0)),
        ],
        out_specs=pl.BlockSpec((1, 1), lambda: (0, 0)),
        out_shape=jax.ShapeDtypeStruct((1, 1), jnp.float32),
    )(step, errors)
    return out[0, 0]
